# baseline (speedup 1.0000x reference)
import functools

import jax
import jax.numpy as jnp
from jax import lax
from jax.experimental import pallas as pl
from jax.experimental.pallas import tpu as pltpu
from jax.experimental.pallas import tpu_sc as plsc

_SEQ = 8192
_DIM = 1024
_NC = 2
_NS = 16
_NW = _NC * _NS
_ROWS_PER_W = _SEQ // _NW


def _copy_body(table_hbm, out_hbm, sem):
    wid = lax.axis_index("s") * _NC + lax.axis_index("c")
    base = wid * _ROWS_PER_W
    pltpu.make_async_copy(
        table_hbm.at[pl.ds(base, _ROWS_PER_W)],
        out_hbm.at[pl.ds(base, _ROWS_PER_W)],
        sem,
    ).start()
    pltpu.make_async_copy(
        table_hbm.at[pl.ds(base, _ROWS_PER_W)],
        out_hbm.at[pl.ds(base, _ROWS_PER_W)],
        sem,
    ).wait()


@jax.jit
def _positional_copy(emb_weight):
    mesh = plsc.VectorSubcoreMesh(core_axis_name="c", subcore_axis_name="s")
    k = functools.partial(
        pl.kernel,
        mesh=mesh,
        out_type=jax.ShapeDtypeStruct((_SEQ, _DIM), jnp.float32),
        scratch_types=[
            pltpu.SemaphoreType.DMA,
        ],
    )(_copy_body)
    return k(emb_weight)


def kernel(x, emb_weight):
    del x
    return _positional_copy(emb_weight)[None, :, :]


# speedup vs baseline: 24.7791x; 24.7791x over previous
import functools

import jax
import jax.numpy as jnp
from jax import lax
from jax.experimental import pallas as pl
from jax.experimental.pallas import tpu as pltpu
from jax.experimental.pallas import tpu_sc as plsc

_SEQ = 8192
_DIM = 1024
_NC = 2
_NS = 16
_NW = _NC * _NS
_ROWS_PER_W = _SEQ // _NW
_CHUNK = 32
_NSTEP = _ROWS_PER_W // _CHUNK
_NBUF = 3


def _copy_body(table_hbm, out_hbm, buf, in_sems, out_sems):
    wid = lax.axis_index("s") * _NC + lax.axis_index("c")
    base = wid * _ROWS_PER_W

    def read(i, slot):
        return pltpu.make_async_copy(
            table_hbm.at[pl.ds(base + i * _CHUNK, _CHUNK)],
            buf.at[slot],
            in_sems.at[slot],
        )

    def write(i, slot):
        return pltpu.make_async_copy(
            buf.at[slot],
            out_hbm.at[pl.ds(base + i * _CHUNK, _CHUNK)],
            out_sems.at[slot],
        )

    for i in range(min(_NBUF, _NSTEP)):
        read(i, i % _NBUF).start()
    for i in range(_NSTEP):
        slot = i % _NBUF
        read(i, slot).wait()
        write(i, slot).start()
        if i + _NBUF < _NSTEP:
            write(i, slot).wait()
            read(i + _NBUF, slot).start()
    for i in range(max(0, _NSTEP - _NBUF), _NSTEP):
        write(i, i % _NBUF).wait()


@functools.partial(jax.jit, static_argnums=())
def _positional_copy(emb_weight):
    mesh = plsc.VectorSubcoreMesh(core_axis_name="c", subcore_axis_name="s")
    k = functools.partial(
        pl.kernel,
        mesh=mesh,
        out_type=jax.ShapeDtypeStruct((_SEQ, _DIM), jnp.float32),
        scratch_types=[
            pltpu.VMEM((_NBUF, _CHUNK, _DIM), jnp.float32),
            pltpu.SemaphoreType.DMA((_NBUF,)),
            pltpu.SemaphoreType.DMA((_NBUF,)),
        ],
    )(_copy_body)
    return k(emb_weight)


def kernel(x, emb_weight):
    del x
    return _positional_copy(emb_weight)[None, :, :]


# speedup vs baseline: 24.8970x; 1.0048x over previous
import functools

import jax
import jax.numpy as jnp
from jax import lax
from jax.experimental import pallas as pl
from jax.experimental.pallas import tpu as pltpu
from jax.experimental.pallas import tpu_sc as plsc

_SEQ = 8192
_DIM = 1024
_NC = 2
_NS = 16
_NW = _NC * _NS
_ROWS_PER_W = _SEQ // _NW
_CHUNK = 16
_NSTEP = _ROWS_PER_W // _CHUNK
_NBUF = 7


def _copy_body(table_hbm, out_hbm, buf, in_sems, out_sems):
    wid = lax.axis_index("s") * _NC + lax.axis_index("c")
    base = wid * _ROWS_PER_W

    def read(i, slot):
        return pltpu.make_async_copy(
            table_hbm.at[pl.ds(base + i * _CHUNK, _CHUNK)],
            buf.at[slot],
            in_sems.at[slot],
        )

    def write(i, slot):
        return pltpu.make_async_copy(
            buf.at[slot],
            out_hbm.at[pl.ds(base + i * _CHUNK, _CHUNK)],
            out_sems.at[slot],
        )

    for i in range(min(_NBUF, _NSTEP)):
        read(i, i % _NBUF).start()
    for i in range(_NSTEP):
        slot = i % _NBUF
        read(i, slot).wait()
        write(i, slot).start()
        if i + _NBUF < _NSTEP:
            write(i, slot).wait()
            read(i + _NBUF, slot).start()
    for i in range(max(0, _NSTEP - _NBUF), _NSTEP):
        write(i, i % _NBUF).wait()


@functools.partial(jax.jit, static_argnums=())
def _positional_copy(emb_weight):
    mesh = plsc.VectorSubcoreMesh(core_axis_name="c", subcore_axis_name="s")
    k = functools.partial(
        pl.kernel,
        mesh=mesh,
        out_type=jax.ShapeDtypeStruct((_SEQ, _DIM), jnp.float32),
        scratch_types=[
            pltpu.VMEM((_NBUF, _CHUNK, _DIM), jnp.float32),
            pltpu.SemaphoreType.DMA((_NBUF,)),
            pltpu.SemaphoreType.DMA((_NBUF,)),
        ],
    )(_copy_body)
    return k(emb_weight)


def kernel(x, emb_weight):
    del x
    return _positional_copy(emb_weight)[None, :, :]


# speedup vs baseline: 24.9767x; 1.0032x over previous
import functools

import jax
import jax.numpy as jnp
from jax import lax
from jax.experimental import pallas as pl
from jax.experimental.pallas import tpu as pltpu
from jax.experimental.pallas import tpu_sc as plsc

_SEQ = 8192
_DIM = 1024
_NC = 2
_NS = 16
_NW = _NC * _NS
_ROWS_PER_W = _SEQ // _NW
_CHUNK = 16
_NSTEP = _ROWS_PER_W // _CHUNK
_NBUF = 7
_LAZY = 3


def _copy_body(table_hbm, out_hbm, buf, in_sems, out_sems):
    wid = lax.axis_index("s") * _NC + lax.axis_index("c")
    base = wid * _ROWS_PER_W

    def read(i):
        return pltpu.make_async_copy(
            table_hbm.at[pl.ds(base + i * _CHUNK, _CHUNK)],
            buf.at[i % _NBUF],
            in_sems.at[i % _NBUF],
        )

    def write(i):
        return pltpu.make_async_copy(
            buf.at[i % _NBUF],
            out_hbm.at[pl.ds(base + i * _CHUNK, _CHUNK)],
            out_sems.at[i % _NBUF],
        )

    waited = [False] * _NSTEP
    for i in range(min(_NBUF, _NSTEP)):
        read(i).start()
    for i in range(_NSTEP):
        read(i).wait()
        write(i).start()
        j = i - _LAZY
        if j >= 0 and j + _NBUF < _NSTEP:
            write(j).wait()
            waited[j] = True
            read(j + _NBUF).start()
    for i in range(_NSTEP):
        if not waited[i]:
            write(i).wait()


@jax.jit
def _positional_copy(emb_weight):
    mesh = plsc.VectorSubcoreMesh(core_axis_name="c", subcore_axis_name="s")
    k = functools.partial(
        pl.kernel,
        mesh=mesh,
        out_type=jax.ShapeDtypeStruct((_SEQ, _DIM), jnp.float32),
        scratch_types=[
            pltpu.VMEM((_NBUF, _CHUNK, _DIM), jnp.float32),
            pltpu.SemaphoreType.DMA((_NBUF,)),
            pltpu.SemaphoreType.DMA((_NBUF,)),
        ],
    )(_copy_body)
    return k(emb_weight)


def kernel(x, emb_weight):
    del x
    return _positional_copy(emb_weight)[None, :, :]
